# baseline (speedup 1.0000x reference)
import functools

import jax
import jax.numpy as jnp
from jax import lax
from jax.experimental import pallas as pl
from jax.experimental.pallas import tpu as pltpu

B = 32


def _head_body(h_ref, gamma_ref, beta_ref, w1_ref, b1_ref, w2_ref, b2_ref, o_ref):
    h = h_ref[...]
    h = gamma_ref[...] * (h * (1.0 / jnp.sqrt(1.0 + 1e-5))) + beta_ref[...]
    h = jnp.maximum(h @ w1_ref[...] + b1_ref[...], 0.0)
    h = h @ w2_ref[...] + b2_ref[...]
    m = jnp.max(h, axis=-1, keepdims=True)
    e = jnp.exp(h - m)
    o_ref[...] = e / jnp.sum(e, axis=-1, keepdims=True)


def _head(h, gamma, beta, w1, b1, w2, b2):
    return pl.pallas_call(
        _head_body,
        out_shape=jax.ShapeDtypeStruct((h.shape[0], w2.shape[1]), jnp.float32),
    )(h, gamma[None, :], beta[None, :], w1, b1[None, :], w2, b2[None, :])


def _gcn(x, row, col, w, W, b, n):
    sl = jnp.arange(n, dtype=row.dtype)
    r = jnp.concatenate([row, sl])
    c = jnp.concatenate([col, sl])
    ww = jnp.concatenate([w, jnp.ones((n,), dtype=w.dtype)])
    deg = jnp.zeros((n,), dtype=w.dtype).at[r].add(ww)
    dinv = jnp.where(deg > 0, jax.lax.rsqrt(jnp.maximum(deg, 1e-12)), 0.0)
    norm = dinv[r] * ww * dinv[c]
    xw = x @ W
    out = jnp.zeros((n, xw.shape[1]), dtype=xw.dtype).at[r].add(norm[:, None] * xw[c])
    return out + b


def kernel(x, edge_index, edge_weight, batch, cover_n, cover_c, edge_index2,
           weights2, batch2, W_in, b_in, W_blk, b_blk, bn_gamma, bn_beta,
           W1, b1, W2, b2):
    N = x.shape[0]
    C = batch2.shape[0]
    x1 = jax.nn.relu(_gcn(x, edge_index[0], edge_index[1], edge_weight, W_in, b_in, N))
    xs = [jax.ops.segment_sum(x1, batch, num_segments=B)]
    m = jax.ops.segment_max(x1, batch, num_segments=B)
    xs.append(jnp.where(jnp.isfinite(m), m, 0.0))
    gathered = x1[cover_n]
    x_add = jax.ops.segment_sum(gathered, cover_c, num_segments=C)
    x_mx = jax.ops.segment_max(gathered, cover_c, num_segments=C)
    x_mx = jnp.where(jnp.isfinite(x_mx), x_mx, 0.0)
    x2 = jnp.concatenate([x_add, x_mx], axis=1)
    x2 = jax.nn.relu(_gcn(x2, edge_index2[0], edge_index2[1], weights2, W_blk, b_blk, C))
    xs.append(jax.ops.segment_sum(x2, batch2, num_segments=B))
    m2 = jax.ops.segment_max(x2, batch2, num_segments=B)
    xs.append(jnp.where(jnp.isfinite(m2), m2, 0.0))
    h = jnp.concatenate(xs, axis=1)
    return _head(h, bn_gamma, bn_beta, W1, b1, W2, b2)


# speedup vs baseline: 2.6045x; 2.6045x over previous
import functools

import jax
import jax.numpy as jnp
from jax import lax
from jax.experimental import pallas as pl
from jax.experimental.pallas import tpu as pltpu
from jax.experimental.pallas import tpu_sc as plsc

B = 32
H = 128


def _matmul(x, W, bs):
    n = x.shape[0]

    def body(x_ref, w_ref, o_ref):
        o_ref[...] = jnp.dot(x_ref[...], w_ref[...],
                             preferred_element_type=jnp.float32)

    return pl.pallas_call(
        body,
        grid=(n // bs,),
        in_specs=[pl.BlockSpec((bs, x.shape[1]), lambda i: (i, 0)),
                  pl.BlockSpec(W.shape, lambda i: (0, 0))],
        out_specs=pl.BlockSpec((bs, W.shape[1]), lambda i: (i, 0)),
        out_shape=jax.ShapeDtypeStruct((n, W.shape[1]), jnp.float32),
    )(x, W)


def _matmul2(xa, xm, Wa, Wb, bs):
    n = xa.shape[0]

    def body(a_ref, m_ref, wa_ref, wb_ref, o_ref):
        o_ref[...] = (jnp.dot(a_ref[...], wa_ref[...],
                              preferred_element_type=jnp.float32)
                      + jnp.dot(m_ref[...], wb_ref[...],
                                preferred_element_type=jnp.float32))

    return pl.pallas_call(
        body,
        grid=(n // bs,),
        in_specs=[pl.BlockSpec((bs, H), lambda i: (i, 0)),
                  pl.BlockSpec((bs, H), lambda i: (i, 0)),
                  pl.BlockSpec((H, H), lambda i: (0, 0)),
                  pl.BlockSpec((H, H), lambda i: (0, 0))],
        out_specs=pl.BlockSpec((bs, H), lambda i: (i, 0)),
        out_shape=jax.ShapeDtypeStruct((n, H), jnp.float32),
    )(xa, xm, Wa, Wb)


def _epilogue(acc, b, n, bs):
    def body(a_ref, b_ref, o_ref):
        o_ref[...] = jnp.maximum(a_ref[0] + a_ref[1] + b_ref[...], 0.0)

    return pl.pallas_call(
        body,
        grid=(n // bs,),
        in_specs=[pl.BlockSpec((2, bs, H), lambda i: (0, i, 0)),
                  pl.BlockSpec((1, H), lambda i: (0, 0))],
        out_specs=pl.BlockSpec((bs, H), lambda i: (i, 0)),
        out_shape=jax.ShapeDtypeStruct((n, H), jnp.float32),
    )(acc, b.reshape(1, H))


def _head(h, gamma, beta, w1, b1, w2, b2):
    def body(h_ref, gamma_ref, beta_ref, w1_ref, b1_ref, w2_ref, b2_ref, o_ref):
        hh = h_ref[...]
        hh = gamma_ref[...] * (hh * (1.0 / jnp.sqrt(1.0 + 1e-5))) + beta_ref[...]
        hh = jnp.maximum(hh @ w1_ref[...] + b1_ref[...], 0.0)
        hh = hh @ w2_ref[...] + b2_ref[...]
        m = jnp.max(hh, axis=-1, keepdims=True)
        e = jnp.exp(hh - m)
        o_ref[...] = e / jnp.sum(e, axis=-1, keepdims=True)

    return pl.pallas_call(
        body,
        out_shape=jax.ShapeDtypeStruct((h.shape[0], w2.shape[1]), jnp.float32),
    )(h, gamma[None, :], beta[None, :], w1, b1[None, :], w2, b2[None, :])


def _newton_rsqrt(d):
    h = 0.5 * d
    i = plsc.bitcast(d, jnp.int32)
    i = jnp.full((16,), 0x5F3759DF, jnp.int32) - lax.shift_right_logical(i, 1)
    y = plsc.bitcast(i, jnp.float32)
    y = y * (1.5 - h * y * y)
    y = y * (1.5 - h * y * y)
    y = y * (1.5 - h * y * y)
    return y


def _edge_agg(xw, rowa, cola, wa, row2d, col2d, n, NP, EP):
    CH, KD, DW = 80, 640, 64
    BLK = 8 * CH
    R = NP // 16
    M = EP // 32
    D = EP // 16
    NBLK = M // BLK
    DCH = D // KD
    NPR = NP // DW
    IR = NPR // 80
    assert M % BLK == 0 and D % KD == 0 and R % CH == 0 and NPR % 80 == 0
    assert (M // CH) % 8 == 0 and (EP // (2 * CH)) % 8 == 0

    mesh = plsc.VectorSubcoreMesh(core_axis_name="c", subcore_axis_name="s")

    @functools.partial(
        pl.kernel,
        out_type=jax.ShapeDtypeStruct((2, NP, H), jnp.float32),
        mesh=mesh,
        scratch_types=[
            pltpu.VMEM((NPR, DW), jnp.float32),
            pltpu.VMEM((CH, H), jnp.float32),
            pltpu.VMEM((BLK,), jnp.int32),
            pltpu.VMEM((BLK,), jnp.int32),
            pltpu.VMEM((BLK,), jnp.float32),
            pltpu.VMEM((8, CH), jnp.int32),
            pltpu.VMEM((8, CH), jnp.int32),
            pltpu.VMEM((KD,), jnp.int32),
            pltpu.VMEM((KD,), jnp.float32),
            pltpu.VMEM((IR, 80), jnp.int32),
            pltpu.VMEM_SHARED((NPR, DW), jnp.float32),
            pltpu.VMEM_SHARED((NP, H), jnp.float32),
            pltpu.SemaphoreType.DMA,
        ],
        compiler_params=pltpu.CompilerParams(needs_layout_passes=False),
    )
    def body(xw_hbm, row_hbm, col_hbm, w_hbm, row2_hbm, col2_hbm,
             acc_hbm, degp, rows, r1v, c1v, w1v, r2v, c2v, rdv, wdv,
             iidx, deg_sh, acc_sh, sem):
        ci = lax.axis_index("c")
        si = lax.axis_index("s")
        zt = jnp.zeros((16,), jnp.float32)
        iota = lax.iota(jnp.int32, 16)

        def zrow(j, carry):
            rows[j // 8, pl.ds(pl.multiple_of((j % 8) * 16, 16), 16)] = zt
            return carry
        lax.fori_loop(0, CH * 8, zrow, 0)

        def zdeg(j, carry):
            degp[j // 4, pl.ds(pl.multiple_of((j % 4) * 16, 16), 16)] = zt
            return carry
        lax.fori_loop(0, NPR * 4, zdeg, 0)

        def zi(j, carry):
            iidx[j // 5, pl.ds(pl.multiple_of((j % 5) * 16, 16), 16)] = (
                j * 16 + iota)
            return carry
        lax.fori_loop(0, IR * 5, zi, 0)

        @pl.when(si == 0)
        def _():
            pltpu.sync_copy(degp, deg_sh)

        for off in range(0, R, CH):
            pltpu.sync_copy(rows, acc_sh.at[pl.ds(si * R + off, CH)])

        def deg_chunk(k, carry):
            base = pl.multiple_of(si * D + k * KD, 8)
            pltpu.sync_copy(row_hbm.at[pl.ds(base, KD)], rdv)
            pltpu.sync_copy(w_hbm.at[pl.ds(base, KD)], wdv)

            def grp(g, c2):
                o = pl.multiple_of(g * 16, 16)
                rr = rdv[pl.ds(o, 16)]
                plsc.addupdate_scatter(
                    degp,
                    [lax.shift_right_logical(rr, 6), rr & (DW - 1)],
                    wdv[pl.ds(o, 16)])
                return c2
            lax.fori_loop(0, KD // 16, grp, 0)
            return carry
        lax.fori_loop(0, DCH, deg_chunk, 0)
        plsc.subcore_barrier()

        for j in range(IR):
            pltpu.sync_copy(degp.at[pl.ds(j * 80, 80)],
                            deg_sh.at[iidx.at[j]], add=True)
        plsc.subcore_barrier()

        pltpu.sync_copy(deg_sh, degp)

        def dj(j, carry):
            o = pl.multiple_of((j % 4) * 16, 16)
            degp[j // 4, pl.ds(o, 16)] = _newton_rsqrt(degp[j // 4, pl.ds(o, 16)])
            return carry
        lax.fori_loop(0, NPR * 4, dj, 0)

        mbase = ci * (EP // 2) + si * M
        mrow = ci * (EP // (2 * CH)) + si * (M // CH)

        def msg_blk(bi, carry):
            b1 = pl.multiple_of(mbase + bi * BLK, 8)
            br = pl.multiple_of(mrow + bi * 8, 8)
            pltpu.sync_copy(row2_hbm.at[pl.ds(br, 8)], r2v)
            pltpu.sync_copy(col2_hbm.at[pl.ds(br, 8)], c2v)
            pltpu.sync_copy(row_hbm.at[pl.ds(b1, BLK)], r1v)
            pltpu.sync_copy(col_hbm.at[pl.ds(b1, BLK)], c1v)
            pltpu.sync_copy(w_hbm.at[pl.ds(b1, BLK)], w1v)

            def chunk(j8, c1):
                pltpu.async_copy(xw_hbm.at[c2v.at[j8]], rows, sem).wait()

                def grp(g, c2):
                    o = pl.multiple_of(j8 * CH + g * 16, 16)
                    rr = r1v[pl.ds(o, 16)]
                    cc = c1v[pl.ds(o, 16)]
                    wv = w1v[pl.ds(o, 16)]
                    dr = plsc.load_gather(
                        degp, [lax.shift_right_logical(rr, 6), rr & (DW - 1)])
                    dc = plsc.load_gather(
                        degp, [lax.shift_right_logical(cc, 6), cc & (DW - 1)])
                    nv = dr * wv * dc
                    rid = g * 16 + iota
                    for f in range(H):
                        cid = jnp.full((16,), f, jnp.int32)
                        v = plsc.load_gather(rows, [rid, cid])
                        plsc.store_scatter(rows, [rid, cid], v * nv)
                    return c2
                lax.fori_loop(0, CH // 16, grp, 0)

                pltpu.sync_copy(rows, acc_sh.at[r2v.at[j8]], add=True)
                return c1
            lax.fori_loop(0, 8, chunk, 0)
            return carry
        lax.fori_loop(0, NBLK, msg_blk, 0)

        plsc.subcore_barrier()
        for off in range(0, R, CH):
            pltpu.sync_copy(acc_sh.at[pl.ds(si * R + off, CH)], rows)
            pltpu.sync_copy(rows, acc_hbm.at[ci, pl.ds(si * R + off, CH)])

    return body(xw, rowa, cola, wa, row2d, col2d)


def _gcn_sc(x_feat_w, row, col, w, n, NP, EP, b):
    e_aug = row.shape[0]
    pad = EP - e_aug
    rowa = jnp.concatenate([row, jnp.zeros((pad,), jnp.int32)])
    cola = jnp.concatenate([col, jnp.zeros((pad,), jnp.int32)])
    wa = jnp.concatenate([w, jnp.zeros((pad,), jnp.float32)])
    row2d = rowa.reshape(-1, 80)
    col2d = cola.reshape(-1, 80)
    acc = _edge_agg(x_feat_w, rowa, cola, wa, row2d, col2d, n, NP, EP)
    return _epilogue(acc, b, n, 1000)


def kernel(x, edge_index, edge_weight, batch, cover_n, cover_c, edge_index2,
           weights2, batch2, W_in, b_in, W_blk, b_blk, bn_gamma, bn_beta,
           W1, b1, W2, b2):
    N = x.shape[0]
    C = batch2.shape[0]
    NP1, EP1 = 10240, 348160
    NP2, EP2 = 5120, 184320

    sl1 = jnp.arange(N, dtype=jnp.int32)
    row1 = jnp.concatenate([edge_index[0], sl1])
    col1 = jnp.concatenate([edge_index[1], sl1])
    w1 = jnp.concatenate([edge_weight, jnp.ones((N,), jnp.float32)])
    xw1 = _matmul(x, W_in, 1000)
    x1 = _gcn_sc(xw1, row1, col1, w1, N, NP1, EP1, b_in)

    xs = [jax.ops.segment_sum(x1, batch, num_segments=B)]
    m = jax.ops.segment_max(x1, batch, num_segments=B)
    xs.append(jnp.where(jnp.isfinite(m), m, 0.0))

    gathered = x1[cover_n]
    x_add = jax.ops.segment_sum(gathered, cover_c, num_segments=C)
    x_mx = jax.ops.segment_max(gathered, cover_c, num_segments=C)
    x_mx = jnp.where(jnp.isfinite(x_mx), x_mx, 0.0)

    sl2 = jnp.arange(C, dtype=jnp.int32)
    row2 = jnp.concatenate([edge_index2[0], sl2])
    col2 = jnp.concatenate([edge_index2[1], sl2])
    w2 = jnp.concatenate([weights2, jnp.ones((C,), jnp.float32)])
    xw2 = _matmul2(x_add, x_mx, W_blk[:H], W_blk[H:], 1000)
    x2 = _gcn_sc(xw2, row2, col2, w2, C, NP2, EP2, b_blk)

    xs.append(jax.ops.segment_sum(x2, batch2, num_segments=B))
    m2 = jax.ops.segment_max(x2, batch2, num_segments=B)
    xs.append(jnp.where(jnp.isfinite(m2), m2, 0.0))

    h = jnp.concatenate(xs, axis=1)
    return _head(h, bn_gamma, bn_beta, W1, b1, W2, b2)
